# Initial kernel scaffold; baseline (speedup 1.0000x reference)
import jax
import jax.numpy as jnp
from jax.experimental import pallas as pl


def kernel(fatoms, fbonds, agraph, bgraph, W_i, W_h, W_o, b_o):
    raise NotImplementedError("write your pallas kernel here")


# speedup vs baseline: 3.9892x; 3.9892x over previous
import functools

import jax
import jax.numpy as jnp
from jax import lax
from jax.experimental import pallas as pl
from jax.experimental.pallas import tpu as pltpu
from jax.experimental.pallas import tpu_sc as plsc

N_ATOMS = 50000
N_BONDS = 200000
MAX_NB = 6
HIDDEN = 128
DEPTH = 4
ATOM_FDIM = 144
IN_BOND = ATOM_FDIM + 14
N_MOLS = 2000
ATOMS_PER_MOL = 25

NUM_SC_CORES = 2
NUM_SC_SUBCORES = 16
NUM_WORKERS = NUM_SC_CORES * NUM_SC_SUBCORES


def _make_gather_sum(n_pad, chunk, steps, table_rows):
  assert NUM_WORKERS * steps * chunk == n_pad
  assert chunk % 8 == 0 and chunk <= 128

  mesh = plsc.VectorSubcoreMesh(core_axis_name="c", subcore_axis_name="s")

  @functools.partial(
      pl.kernel,
      out_type=jax.ShapeDtypeStruct((n_pad, HIDDEN), jnp.float32),
      mesh=mesh,
      scratch_types=[
          pltpu.VMEM((MAX_NB, chunk), jnp.int32),
          pltpu.VMEM((chunk, HIDDEN), jnp.float32),
          pltpu.SemaphoreType.DMA,
      ],
  )
  def gather_sum(table_hbm, idx_hbm, out_hbm, idx_v, acc_v, sem):
    wid = lax.axis_index("s") * NUM_SC_CORES + lax.axis_index("c")

    def step(t, carry):
      g = wid * steps + t
      base = g * chunk
      pltpu.sync_copy(idx_hbm.at[g], idx_v)
      pltpu.async_copy(table_hbm.at[idx_v.at[0]], acc_v, sem).wait()
      cps = [
          pltpu.async_copy(table_hbm.at[idx_v.at[k]], acc_v, sem, add=True)
          for k in range(1, MAX_NB)
      ]
      for cp in cps:
        cp.wait()
      pltpu.sync_copy(acc_v, out_hbm.at[pl.ds(base, chunk)])
      return carry

    lax.fori_loop(0, steps, step, 0)

  return gather_sum


def _pad_indices(graph, n_pad, chunk):
  n = graph.shape[0]
  g = jnp.pad(graph.astype(jnp.int32), ((0, n_pad - n), (0, 0)))
  return g.T.reshape(MAX_NB, n_pad // chunk, chunk).transpose(1, 0, 2)


def _tc_input_proj(fbonds, W_i):
  blk = 2000

  def body(x_ref, w_ref, b_ref, m_ref):
    b = jnp.dot(x_ref[...], w_ref[...], preferred_element_type=jnp.float32)
    b_ref[...] = b
    m_ref[...] = jnp.maximum(b, 0.0)

  return pl.pallas_call(
      body,
      grid=(N_BONDS // blk,),
      in_specs=[
          pl.BlockSpec((blk, IN_BOND), lambda i: (i, 0)),
          pl.BlockSpec((IN_BOND, HIDDEN), lambda i: (0, 0)),
      ],
      out_specs=[
          pl.BlockSpec((blk, HIDDEN), lambda i: (i, 0)),
          pl.BlockSpec((blk, HIDDEN), lambda i: (i, 0)),
      ],
      out_shape=[
          jax.ShapeDtypeStruct((N_BONDS, HIDDEN), jnp.float32),
          jax.ShapeDtypeStruct((N_BONDS, HIDDEN), jnp.float32),
      ],
  )(fbonds, W_i)


def _tc_msg_update(binput, nei, W_h):
  blk = 2000

  def body(b_ref, n_ref, w_ref, o_ref):
    o_ref[...] = jnp.maximum(
        b_ref[...]
        + jnp.dot(n_ref[...], w_ref[...], preferred_element_type=jnp.float32),
        0.0,
    )

  return pl.pallas_call(
      body,
      grid=(N_BONDS // blk,),
      in_specs=[
          pl.BlockSpec((blk, HIDDEN), lambda i: (i, 0)),
          pl.BlockSpec((blk, HIDDEN), lambda i: (i, 0)),
          pl.BlockSpec((HIDDEN, HIDDEN), lambda i: (0, 0)),
      ],
      out_specs=pl.BlockSpec((blk, HIDDEN), lambda i: (i, 0)),
      out_shape=jax.ShapeDtypeStruct((N_BONDS, HIDDEN), jnp.float32),
  )(binput, nei, W_h)


def _tc_readout(fatoms, anei, W_oa, W_on, b_o, seg):
  blk = 1000
  mols_per_blk = blk // ATOMS_PER_MOL

  def body(fa_ref, an_ref, woa_ref, won_ref, bo_ref, seg_ref, o_ref):
    h = (
        jnp.dot(fa_ref[...], woa_ref[...], preferred_element_type=jnp.float32)
        + jnp.dot(an_ref[...], won_ref[...], preferred_element_type=jnp.float32)
        + bo_ref[...]
    )
    h = jnp.maximum(h, 0.0)
    o_ref[...] = jnp.dot(seg_ref[...], h, preferred_element_type=jnp.float32)

  return pl.pallas_call(
      body,
      grid=(N_ATOMS // blk,),
      in_specs=[
          pl.BlockSpec((blk, ATOM_FDIM), lambda i: (i, 0)),
          pl.BlockSpec((blk, HIDDEN), lambda i: (i, 0)),
          pl.BlockSpec((ATOM_FDIM, HIDDEN), lambda i: (0, 0)),
          pl.BlockSpec((HIDDEN, HIDDEN), lambda i: (0, 0)),
          pl.BlockSpec((1, HIDDEN), lambda i: (0, 0)),
          pl.BlockSpec((mols_per_blk, blk), lambda i: (0, 0)),
      ],
      out_specs=pl.BlockSpec((mols_per_blk, HIDDEN), lambda i: (i, 0)),
      out_shape=jax.ShapeDtypeStruct((N_MOLS, HIDDEN), jnp.float32),
  )(fatoms, anei, W_oa, W_on, b_o, seg)


BOND_CHUNK = 128
BOND_STEPS = 49
BOND_PAD = NUM_WORKERS * BOND_STEPS * BOND_CHUNK

ATOM_CHUNK = 112
ATOM_STEPS = 14
ATOM_PAD = NUM_WORKERS * ATOM_STEPS * ATOM_CHUNK

_bond_gather = _make_gather_sum(BOND_PAD, BOND_CHUNK, BOND_STEPS, N_BONDS)
_atom_gather = _make_gather_sum(ATOM_PAD, ATOM_CHUNK, ATOM_STEPS, N_BONDS)


def kernel(fatoms, fbonds, agraph, bgraph, W_i, W_h, W_o, b_o):
  bidx = _pad_indices(bgraph, BOND_PAD, BOND_CHUNK)
  aidx = _pad_indices(agraph, ATOM_PAD, ATOM_CHUNK)

  binput, message = _tc_input_proj(fbonds, W_i)

  for _ in range(DEPTH - 1):
    nei = _bond_gather(message, bidx)[:N_BONDS]
    message = _tc_msg_update(binput, nei, W_h)

  anei = _atom_gather(message, aidx)[:N_ATOMS]

  W_oa = W_o[:ATOM_FDIM]
  W_on = W_o[ATOM_FDIM:]
  blk = 1000
  mols_per_blk = blk // ATOMS_PER_MOL
  seg = jnp.kron(
      jnp.eye(mols_per_blk, dtype=jnp.float32),
      jnp.full((1, ATOMS_PER_MOL), 1.0 / ATOMS_PER_MOL, dtype=jnp.float32),
  )
  return _tc_readout(fatoms, anei, W_oa, W_on, b_o.reshape(1, HIDDEN), seg)


# speedup vs baseline: 4.4261x; 1.1095x over previous
import functools

import jax
import jax.numpy as jnp
from jax import lax
from jax.experimental import pallas as pl
from jax.experimental.pallas import tpu as pltpu
from jax.experimental.pallas import tpu_sc as plsc

N_ATOMS = 50000
N_BONDS = 200000
MAX_NB = 6
HIDDEN = 128
DEPTH = 4
ATOM_FDIM = 144
IN_BOND = ATOM_FDIM + 14
N_MOLS = 2000
ATOMS_PER_MOL = 25

NUM_SC_CORES = 2
NUM_SC_SUBCORES = 16
NUM_WORKERS = NUM_SC_CORES * NUM_SC_SUBCORES


NBUF = 4


def _make_gather_sum(n_pad, chunk, steps, table_rows):
  assert NUM_WORKERS * steps * chunk == n_pad
  assert chunk % 8 == 0 and chunk <= 128

  mesh = plsc.VectorSubcoreMesh(core_axis_name="c", subcore_axis_name="s")

  @functools.partial(
      pl.kernel,
      out_type=jax.ShapeDtypeStruct((n_pad, HIDDEN), jnp.float32),
      mesh=mesh,
      scratch_types=[
          pltpu.VMEM((NBUF, MAX_NB, chunk), jnp.int32),
          pltpu.VMEM((NBUF, chunk, HIDDEN), jnp.float32),
          pltpu.SemaphoreType.DMA((NBUF,)),
          pltpu.SemaphoreType.DMA((NBUF,)),
          pltpu.SemaphoreType.DMA((NBUF,)),
          pltpu.SemaphoreType.DMA((NBUF,)),
      ],
  )
  def gather_sum(table_hbm, idx_hbm, out_hbm, idx_v, acc_v, isem, gsem, asem,
                 osem):
    wid = lax.axis_index("s") * NUM_SC_CORES + lax.axis_index("c")
    step0 = wid * steps

    def idx_cp(t, p):
      return pltpu.async_copy(idx_hbm.at[step0 + t], idx_v.at[p], isem.at[p])

    def stage_a(t, p):
      pltpu.make_async_copy(idx_hbm.at[step0 + t], idx_v.at[p],
                            isem.at[p]).wait()
      pltpu.async_copy(table_hbm.at[idx_v.at[p, 0]], acc_v.at[p], gsem.at[p])

    def stage_b(t, p):
      pltpu.make_async_copy(table_hbm.at[idx_v.at[p, 0]], acc_v.at[p],
                            gsem.at[p]).wait()
      for k in range(1, MAX_NB):
        pltpu.async_copy(table_hbm.at[idx_v.at[p, k]], acc_v.at[p], asem.at[p],
                         add=True)

    def stage_c(t, p):
      for _ in range(MAX_NB - 1):
        pltpu.make_async_copy(table_hbm.at[idx_v.at[p, 1]], acc_v.at[p],
                              asem.at[p]).wait()
      pltpu.async_copy(acc_v.at[p],
                       out_hbm.at[pl.ds((step0 + t) * chunk, chunk)],
                       osem.at[p])

    def stage_d(t, p):
      pltpu.make_async_copy(acc_v.at[p],
                            out_hbm.at[pl.ds((step0 + t) * chunk, chunk)],
                            osem.at[p]).wait()

    for t in range(min(NBUF, steps)):
      idx_cp(t, t)
    stage_a(0, 0)
    if steps > 1:
      stage_b(0, 0)
      stage_a(1, 1)
    if steps > 2:
      stage_c(0, 0)
      stage_b(1, 1)
      stage_a(2, 2)

    def body(t, carry):
      p = lax.rem(t, NBUF)
      pm1 = lax.rem(t - 1, NBUF)
      pm2 = lax.rem(t - 2, NBUF)
      pm3 = lax.rem(t - 3, NBUF)
      stage_a(t, p)
      stage_b(t - 1, pm1)
      stage_c(t - 2, pm2)
      stage_d(t - 3, pm3)
      idx_cp(t + 1, pm3)
      return carry

    lax.fori_loop(3, steps - 1, body, 0)

    t = steps - 1
    stage_a(t, t % NBUF)
    stage_b(t - 1, (t - 1) % NBUF)
    stage_c(t - 2, (t - 2) % NBUF)
    stage_d(t - 3, (t - 3) % NBUF)
    stage_b(t, t % NBUF)
    stage_c(t - 1, (t - 1) % NBUF)
    stage_d(t - 2, (t - 2) % NBUF)
    stage_c(t, t % NBUF)
    stage_d(t - 1, (t - 1) % NBUF)
    stage_d(t, t % NBUF)

  return gather_sum


def _pad_indices(graph, n_pad, chunk):
  n = graph.shape[0]
  g = jnp.pad(graph.astype(jnp.int32), ((0, n_pad - n), (0, 0)))
  return g.T.reshape(MAX_NB, n_pad // chunk, chunk).transpose(1, 0, 2)


def _tc_input_proj(fbonds, W_i):
  blk = 2000

  def body(x_ref, w_ref, b_ref, m_ref):
    b = jnp.dot(x_ref[...], w_ref[...], preferred_element_type=jnp.float32)
    b_ref[...] = b
    m_ref[...] = jnp.maximum(b, 0.0)

  return pl.pallas_call(
      body,
      grid=(N_BONDS // blk,),
      in_specs=[
          pl.BlockSpec((blk, IN_BOND), lambda i: (i, 0)),
          pl.BlockSpec((IN_BOND, HIDDEN), lambda i: (0, 0)),
      ],
      out_specs=[
          pl.BlockSpec((blk, HIDDEN), lambda i: (i, 0)),
          pl.BlockSpec((blk, HIDDEN), lambda i: (i, 0)),
      ],
      out_shape=[
          jax.ShapeDtypeStruct((N_BONDS, HIDDEN), jnp.float32),
          jax.ShapeDtypeStruct((N_BONDS, HIDDEN), jnp.float32),
      ],
  )(fbonds, W_i)


def _tc_msg_update(binput, nei, W_h):
  blk = 2000

  def body(b_ref, n_ref, w_ref, o_ref):
    o_ref[...] = jnp.maximum(
        b_ref[...]
        + jnp.dot(n_ref[...], w_ref[...], preferred_element_type=jnp.float32),
        0.0,
    )

  return pl.pallas_call(
      body,
      grid=(N_BONDS // blk,),
      in_specs=[
          pl.BlockSpec((blk, HIDDEN), lambda i: (i, 0)),
          pl.BlockSpec((blk, HIDDEN), lambda i: (i, 0)),
          pl.BlockSpec((HIDDEN, HIDDEN), lambda i: (0, 0)),
      ],
      out_specs=pl.BlockSpec((blk, HIDDEN), lambda i: (i, 0)),
      out_shape=jax.ShapeDtypeStruct((N_BONDS, HIDDEN), jnp.float32),
  )(binput, nei, W_h)


def _tc_readout(fatoms, anei, W_oa, W_on, b_o, seg):
  blk = 1000
  mols_per_blk = blk // ATOMS_PER_MOL

  def body(fa_ref, an_ref, woa_ref, won_ref, bo_ref, seg_ref, o_ref):
    h = (
        jnp.dot(fa_ref[...], woa_ref[...], preferred_element_type=jnp.float32)
        + jnp.dot(an_ref[...], won_ref[...], preferred_element_type=jnp.float32)
        + bo_ref[...]
    )
    h = jnp.maximum(h, 0.0)
    o_ref[...] = jnp.dot(seg_ref[...], h, preferred_element_type=jnp.float32)

  return pl.pallas_call(
      body,
      grid=(N_ATOMS // blk,),
      in_specs=[
          pl.BlockSpec((blk, ATOM_FDIM), lambda i: (i, 0)),
          pl.BlockSpec((blk, HIDDEN), lambda i: (i, 0)),
          pl.BlockSpec((ATOM_FDIM, HIDDEN), lambda i: (0, 0)),
          pl.BlockSpec((HIDDEN, HIDDEN), lambda i: (0, 0)),
          pl.BlockSpec((1, HIDDEN), lambda i: (0, 0)),
          pl.BlockSpec((mols_per_blk, blk), lambda i: (0, 0)),
      ],
      out_specs=pl.BlockSpec((mols_per_blk, HIDDEN), lambda i: (i, 0)),
      out_shape=jax.ShapeDtypeStruct((N_MOLS, HIDDEN), jnp.float32),
  )(fatoms, anei, W_oa, W_on, b_o, seg)


BOND_CHUNK = 128
BOND_STEPS = 49
BOND_PAD = NUM_WORKERS * BOND_STEPS * BOND_CHUNK

ATOM_CHUNK = 112
ATOM_STEPS = 14
ATOM_PAD = NUM_WORKERS * ATOM_STEPS * ATOM_CHUNK

_bond_gather = _make_gather_sum(BOND_PAD, BOND_CHUNK, BOND_STEPS, N_BONDS)
_atom_gather = _make_gather_sum(ATOM_PAD, ATOM_CHUNK, ATOM_STEPS, N_BONDS)


def kernel(fatoms, fbonds, agraph, bgraph, W_i, W_h, W_o, b_o):
  bidx = _pad_indices(bgraph, BOND_PAD, BOND_CHUNK)
  aidx = _pad_indices(agraph, ATOM_PAD, ATOM_CHUNK)

  binput, message = _tc_input_proj(fbonds, W_i)

  for _ in range(DEPTH - 1):
    nei = _bond_gather(message, bidx)[:N_BONDS]
    message = _tc_msg_update(binput, nei, W_h)

  anei = _atom_gather(message, aidx)[:N_ATOMS]

  W_oa = W_o[:ATOM_FDIM]
  W_on = W_o[ATOM_FDIM:]
  blk = 1000
  mols_per_blk = blk // ATOMS_PER_MOL
  seg = jnp.kron(
      jnp.eye(mols_per_blk, dtype=jnp.float32),
      jnp.full((1, ATOMS_PER_MOL), 1.0 / ATOMS_PER_MOL, dtype=jnp.float32),
  )
  return _tc_readout(fatoms, anei, W_oa, W_on, b_o.reshape(1, HIDDEN), seg)


# speedup vs baseline: 4.9767x; 1.1244x over previous
import functools

import jax
import jax.numpy as jnp
from jax import lax
from jax.experimental import pallas as pl
from jax.experimental.pallas import tpu as pltpu
from jax.experimental.pallas import tpu_sc as plsc

N_ATOMS = 50000
N_BONDS = 200000
MAX_NB = 6
HIDDEN = 128
DEPTH = 4
ATOM_FDIM = 144
IN_BOND = ATOM_FDIM + 14
N_MOLS = 2000
ATOMS_PER_MOL = 25

NUM_SC_CORES = 2
NUM_SC_SUBCORES = 16
NUM_WORKERS = NUM_SC_CORES * NUM_SC_SUBCORES


NBUF = 4


def _make_gather_sum(n_pad, chunk, steps, table_rows):
  assert NUM_WORKERS * steps * chunk == n_pad
  assert chunk % 8 == 0 and chunk <= 128

  mesh = plsc.VectorSubcoreMesh(core_axis_name="c", subcore_axis_name="s")

  @functools.partial(
      pl.kernel,
      out_type=jax.ShapeDtypeStruct((n_pad, HIDDEN), jnp.float32),
      mesh=mesh,
      scratch_types=[
          pltpu.VMEM((NBUF, MAX_NB, chunk), jnp.int32),
          pltpu.VMEM((NBUF, chunk, HIDDEN), jnp.float32),
          pltpu.SemaphoreType.DMA((NBUF,)),
          pltpu.SemaphoreType.DMA((NBUF,)),
          pltpu.SemaphoreType.DMA((NBUF,)),
          pltpu.SemaphoreType.DMA((NBUF,)),
      ],
  )
  def gather_sum(table_hbm, idx_hbm, out_hbm, idx_v, acc_v, isem, gsem, asem,
                 osem):
    wid = lax.axis_index("s") * NUM_SC_CORES + lax.axis_index("c")
    step0 = wid * steps

    def idx_cp(t, p):
      return pltpu.async_copy(idx_hbm.at[step0 + t], idx_v.at[p], isem.at[p])

    def stage_a(t, p):
      pltpu.make_async_copy(idx_hbm.at[step0 + t], idx_v.at[p],
                            isem.at[p]).wait()
      pltpu.async_copy(table_hbm.at[idx_v.at[p, 0]], acc_v.at[p], gsem.at[p])

    def stage_b(t, p):
      pltpu.make_async_copy(table_hbm.at[idx_v.at[p, 0]], acc_v.at[p],
                            gsem.at[p]).wait()
      for k in range(1, MAX_NB):
        pltpu.async_copy(table_hbm.at[idx_v.at[p, k]], acc_v.at[p], asem.at[p],
                         add=True)

    def stage_c(t, p):
      for _ in range(MAX_NB - 1):
        pltpu.make_async_copy(table_hbm.at[idx_v.at[p, 1]], acc_v.at[p],
                              asem.at[p]).wait()
      pltpu.async_copy(acc_v.at[p],
                       out_hbm.at[pl.ds((step0 + t) * chunk, chunk)],
                       osem.at[p])

    def stage_d(t, p):
      pltpu.make_async_copy(acc_v.at[p],
                            out_hbm.at[pl.ds((step0 + t) * chunk, chunk)],
                            osem.at[p]).wait()

    for t in range(min(NBUF, steps)):
      idx_cp(t, t)
    stage_a(0, 0)
    if steps > 1:
      stage_b(0, 0)
      stage_a(1, 1)
    if steps > 2:
      stage_c(0, 0)
      stage_b(1, 1)
      stage_a(2, 2)

    def body(t, carry):
      p = lax.rem(t, NBUF)
      pm1 = lax.rem(t - 1, NBUF)
      pm2 = lax.rem(t - 2, NBUF)
      pm3 = lax.rem(t - 3, NBUF)
      stage_a(t, p)
      stage_b(t - 1, pm1)
      stage_c(t - 2, pm2)
      stage_d(t - 3, pm3)
      idx_cp(t + 1, pm3)
      return carry

    lax.fori_loop(3, steps - 1, body, 0)

    t = steps - 1
    stage_a(t, t % NBUF)
    stage_b(t - 1, (t - 1) % NBUF)
    stage_c(t - 2, (t - 2) % NBUF)
    stage_d(t - 3, (t - 3) % NBUF)
    stage_b(t, t % NBUF)
    stage_c(t - 1, (t - 1) % NBUF)
    stage_d(t - 2, (t - 2) % NBUF)
    stage_c(t, t % NBUF)
    stage_d(t - 1, (t - 1) % NBUF)
    stage_d(t, t % NBUF)

  return gather_sum


def _pad_indices(graph, n_pad, chunk):
  n = graph.shape[0]
  g = jnp.pad(graph.astype(jnp.int32), ((0, n_pad - n), (0, 0)))
  return g.T.reshape(MAX_NB, n_pad // chunk, chunk).transpose(1, 0, 2)


def _tc_input_proj(fbonds, W_i):
  blk = 2000

  def body(x_ref, w_ref, b_ref, m_ref):
    b = jnp.dot(x_ref[...], w_ref[...], preferred_element_type=jnp.float32)
    b_ref[...] = b
    m_ref[...] = jnp.maximum(b, 0.0)

  return pl.pallas_call(
      body,
      grid=(N_BONDS // blk,),
      in_specs=[
          pl.BlockSpec((blk, IN_BOND), lambda i: (i, 0)),
          pl.BlockSpec((IN_BOND, HIDDEN), lambda i: (0, 0)),
      ],
      out_specs=[
          pl.BlockSpec((blk, HIDDEN), lambda i: (i, 0)),
          pl.BlockSpec((blk, HIDDEN), lambda i: (i, 0)),
      ],
      out_shape=[
          jax.ShapeDtypeStruct((N_BONDS, HIDDEN), jnp.float32),
          jax.ShapeDtypeStruct((N_BONDS, HIDDEN), jnp.float32),
      ],
  )(fbonds, W_i)


def _tc_msg_update(binput, nei_padded, W_h):
  blk = 2000

  def body(b_ref, n_ref, w_ref, o_ref):
    o_ref[...] = jnp.maximum(
        b_ref[...]
        + jnp.dot(n_ref[...], w_ref[...], preferred_element_type=jnp.float32),
        0.0,
    )

  return pl.pallas_call(
      body,
      grid=(N_BONDS // blk,),
      in_specs=[
          pl.BlockSpec((blk, HIDDEN), lambda i: (i, 0)),
          pl.BlockSpec((blk, HIDDEN), lambda i: (i, 0)),
          pl.BlockSpec((HIDDEN, HIDDEN), lambda i: (0, 0)),
      ],
      out_specs=pl.BlockSpec((blk, HIDDEN), lambda i: (i, 0)),
      out_shape=jax.ShapeDtypeStruct((N_BONDS, HIDDEN), jnp.float32),
  )(binput, nei_padded, W_h)


def _tc_readout(fatoms, anei, W_oa, W_on, b_o, seg):
  blk = 1000
  mols_per_blk = blk // ATOMS_PER_MOL

  def body(fa_ref, an_ref, woa_ref, won_ref, bo_ref, seg_ref, o_ref):
    h = (
        jnp.dot(fa_ref[...], woa_ref[...], preferred_element_type=jnp.float32)
        + jnp.dot(an_ref[...], won_ref[...], preferred_element_type=jnp.float32)
        + bo_ref[...]
    )
    h = jnp.maximum(h, 0.0)
    o_ref[...] = jnp.dot(seg_ref[...], h, preferred_element_type=jnp.float32)

  return pl.pallas_call(
      body,
      grid=(N_ATOMS // blk,),
      in_specs=[
          pl.BlockSpec((blk, ATOM_FDIM), lambda i: (i, 0)),
          pl.BlockSpec((blk, HIDDEN), lambda i: (i, 0)),
          pl.BlockSpec((ATOM_FDIM, HIDDEN), lambda i: (0, 0)),
          pl.BlockSpec((HIDDEN, HIDDEN), lambda i: (0, 0)),
          pl.BlockSpec((1, HIDDEN), lambda i: (0, 0)),
          pl.BlockSpec((mols_per_blk, blk), lambda i: (0, 0)),
      ],
      out_specs=pl.BlockSpec((mols_per_blk, HIDDEN), lambda i: (i, 0)),
      out_shape=jax.ShapeDtypeStruct((N_MOLS, HIDDEN), jnp.float32),
  )(fatoms, anei, W_oa, W_on, b_o, seg)


BOND_CHUNK = 128
BOND_STEPS = 49
BOND_PAD = NUM_WORKERS * BOND_STEPS * BOND_CHUNK

ATOM_CHUNK = 112
ATOM_STEPS = 14
ATOM_PAD = NUM_WORKERS * ATOM_STEPS * ATOM_CHUNK

_bond_gather = _make_gather_sum(BOND_PAD, BOND_CHUNK, BOND_STEPS, N_BONDS)
_atom_gather = _make_gather_sum(ATOM_PAD, ATOM_CHUNK, ATOM_STEPS, N_BONDS)


def kernel(fatoms, fbonds, agraph, bgraph, W_i, W_h, W_o, b_o):
  bidx = _pad_indices(bgraph, BOND_PAD, BOND_CHUNK)
  aidx = _pad_indices(agraph, ATOM_PAD, ATOM_CHUNK)

  binput, message = _tc_input_proj(fbonds, W_i)

  for _ in range(DEPTH - 1):
    nei = _bond_gather(message, bidx)
    message = _tc_msg_update(binput, nei, W_h)

  anei = _atom_gather(message, aidx)

  W_oa = W_o[:ATOM_FDIM]
  W_on = W_o[ATOM_FDIM:]
  blk = 1000
  mols_per_blk = blk // ATOMS_PER_MOL
  seg = jnp.kron(
      jnp.eye(mols_per_blk, dtype=jnp.float32),
      jnp.full((1, ATOMS_PER_MOL), 1.0 / ATOMS_PER_MOL, dtype=jnp.float32),
  )
  return _tc_readout(fatoms, anei, W_oa, W_on, b_o.reshape(1, HIDDEN), seg)
